# Initial kernel scaffold; baseline (speedup 1.0000x reference)
import functools

import jax
import jax.numpy as jnp
from jax import lax
from jax.experimental import pallas as pl
from jax.experimental.pallas import tpu as pltpu
from jax.experimental.pallas import tpu_sc as plsc

N = 10000
E = 320000
H = 128
NC = 2
NS = 16
NW = NC * NS
EPW = E // NW
K = 80
WINS = EPW // K
RPS = N // NS
ZR = 125


def _sc_segsum(g, src, dst):
    mesh = plsc.VectorSubcoreMesh(core_axis_name="c", subcore_axis_name="s")

    @functools.partial(
        pl.kernel,
        out_type=jax.ShapeDtypeStruct((NC, N, H), jnp.float32),
        mesh=mesh,
        scratch_types=[
            pltpu.VMEM((K,), jnp.int32),
            pltpu.VMEM((K,), jnp.int32),
            pltpu.VMEM((K, H), jnp.float32),
            pltpu.VMEM((ZR, H), jnp.float32),
            pltpu.VMEM_SHARED((N, H), jnp.float32),
            pltpu.SemaphoreType.DMA,
        ],
    )
    def k(g_hbm, src_hbm, dst_hbm, out_hbm, src_v, dst_v, rows_v, zbuf, acc,
          sem):
        cid = lax.axis_index("c")
        sid = lax.axis_index("s")
        wid = sid * NC + cid

        zero16 = jnp.zeros((16,), jnp.float32)

        @pl.loop(0, ZR)
        def _(i):
            for j in range(H // 16):
                zbuf[i, pl.ds(j * 16, 16)] = zero16

        row0 = sid * RPS
        for j in range(RPS // ZR):
            pltpu.sync_copy(zbuf, acc.at[pl.ds(row0 + j * ZR, ZR)])
        plsc.subcore_barrier()

        base0 = wid * EPW

        @pl.loop(0, WINS)
        def _(w):
            base = base0 + w * K
            pltpu.sync_copy(src_hbm.at[pl.ds(base, K)], src_v)
            pltpu.sync_copy(dst_hbm.at[pl.ds(base, K)], dst_v)
            pltpu.async_copy(g_hbm.at[src_v], rows_v, sem).wait()
            pltpu.sync_copy(rows_v, acc.at[dst_v], add=True)

        plsc.subcore_barrier()
        pltpu.sync_copy(acc.at[pl.ds(row0, RPS)],
                        out_hbm.at[cid].at[pl.ds(row0, RPS)])

    return k(g, src, dst)


def _dot_t(a, b):
    return lax.dot_general(a, b, (((1,), (1,)), ((), ())),
                           preferred_element_type=jnp.float32)


def _tc_pre(x, wr, wo, br):
    def body(x_ref, wr_ref, wo_ref, br_ref, g_ref, r_ref):
        xv = x_ref[...]
        g_ref[...] = _dot_t(xv, wr_ref[...])
        r_ref[...] = _dot_t(xv, wo_ref[...]) + br_ref[...]

    return pl.pallas_call(
        body,
        out_shape=(jax.ShapeDtypeStruct((N, H), jnp.float32),
                   jax.ShapeDtypeStruct((N, H), jnp.float32)),
    )(x, wr, wo, br.reshape(1, H))


def _tc_mid(p, r_prev, wr, wo, br):
    def body(p_ref, rp_ref, wr_ref, wo_ref, br_ref, g_ref, r_ref):
        h = jnp.maximum(p_ref[0] + p_ref[1] + rp_ref[...], 0.0)
        g_ref[...] = _dot_t(h, wr_ref[...])
        r_ref[...] = _dot_t(h, wo_ref[...]) + br_ref[...]

    return pl.pallas_call(
        body,
        out_shape=(jax.ShapeDtypeStruct((N, H), jnp.float32),
                   jax.ShapeDtypeStruct((N, H), jnp.float32)),
    )(p, r_prev, wr, wo, br.reshape(1, H))


def _tc_fin(p, r_prev, lin_w, lin_b):
    def body(p_ref, rp_ref, lw_ref, lb_ref, o_ref):
        h = jnp.maximum(p_ref[0] + p_ref[1] + rp_ref[...], 0.0)
        emb = jnp.sum(h, axis=0, keepdims=True) * (1.0 / N)
        o_ref[...] = _dot_t(emb, lw_ref[...]) + lb_ref[...]

    c = lin_w.shape[0]
    return pl.pallas_call(
        body,
        out_shape=jax.ShapeDtypeStruct((1, c), jnp.float32),
    )(p, r_prev, lin_w, lin_b.reshape(1, c))


def kernel(x, edge_index, W_rel1, b_rel1, W_root1, W_rel2, b_rel2, W_root2,
           W_rel3, b_rel3, W_root3, lin_W, lin_b):
    src = edge_index[0]
    dst = edge_index[1]

    g1, r1 = _tc_pre(x, W_rel1, W_root1, b_rel1)
    p1 = _sc_segsum(g1, src, dst)
    g2, r2 = _tc_mid(p1, r1, W_rel2, W_root2, b_rel2)
    p2 = _sc_segsum(g2, src, dst)
    g3, r3 = _tc_mid(p2, r2, W_rel3, W_root3, b_rel3)
    p3 = _sc_segsum(g3, src, dst)
    return _tc_fin(p3, r3, lin_W, lin_b)


# speedup vs baseline: 5.0898x; 5.0898x over previous
import functools

import jax
import jax.numpy as jnp
from jax import lax
from jax.experimental import pallas as pl
from jax.experimental.pallas import tpu as pltpu
from jax.experimental.pallas import tpu_sc as plsc

N = 10000
E = 320000
H = 128
NC = 2
NS = 16
NW = NC * NS
EPW = E // NW
K = 80
WINS = EPW // K
NP = 10240
RPS = NP // NS
ZR = 128


def _sc_segsum(g, src, dst):
    mesh = plsc.VectorSubcoreMesh(core_axis_name="c", subcore_axis_name="s")

    @functools.partial(
        pl.kernel,
        out_type=jax.ShapeDtypeStruct((NC, NP, H), jnp.float32),
        mesh=mesh,
        scratch_types=[
            pltpu.VMEM((K,), jnp.int32),
            pltpu.VMEM((K,), jnp.int32),
            pltpu.VMEM((K, H), jnp.float32),
            pltpu.VMEM((ZR, H), jnp.float32),
            pltpu.VMEM_SHARED((NP, H), jnp.float32),
            pltpu.SemaphoreType.DMA,
        ],
    )
    def k(g_hbm, src_hbm, dst_hbm, out_hbm, src_v, dst_v, rows_v, zbuf, acc,
          sem):
        cid = lax.axis_index("c")
        sid = lax.axis_index("s")
        wid = sid * NC + cid

        zero16 = jnp.zeros((16,), jnp.float32)

        @pl.loop(0, ZR)
        def _(i):
            for j in range(H // 16):
                zbuf[i, pl.ds(j * 16, 16)] = zero16

        row0 = sid * RPS
        for j in range(RPS // ZR):
            pltpu.sync_copy(zbuf, acc.at[pl.ds(row0 + j * ZR, ZR)])
        plsc.subcore_barrier()

        base0 = wid * EPW

        @pl.loop(0, WINS)
        def _(w):
            base = base0 + w * K
            pltpu.sync_copy(src_hbm.at[pl.ds(base, K)], src_v)
            pltpu.sync_copy(dst_hbm.at[pl.ds(base, K)], dst_v)
            pltpu.async_copy(g_hbm.at[src_v], rows_v, sem).wait()
            pltpu.sync_copy(rows_v, acc.at[dst_v], add=True)

        plsc.subcore_barrier()
        pltpu.sync_copy(acc.at[pl.ds(row0, RPS)],
                        out_hbm.at[cid].at[pl.ds(row0, RPS)])

    return k(g, src, dst)


def _dot_t(a, b):
    return lax.dot_general(a, b, (((1,), (1,)), ((), ())),
                           preferred_element_type=jnp.float32)


def _tc_pre(x, wr, wo, br):
    def body(x_ref, wr_ref, wo_ref, br_ref, g_ref, r_ref):
        xv = x_ref[...]
        g_ref[...] = _dot_t(xv, wr_ref[...])
        r_ref[...] = _dot_t(xv, wo_ref[...]) + br_ref[...]

    return pl.pallas_call(
        body,
        out_shape=(jax.ShapeDtypeStruct((N, H), jnp.float32),
                   jax.ShapeDtypeStruct((N, H), jnp.float32)),
    )(x, wr, wo, br.reshape(1, H))


def _tc_mid(p, r_prev, wr, wo, br):
    def body(p_ref, rp_ref, wr_ref, wo_ref, br_ref, g_ref, r_ref):
        h = jnp.maximum(p_ref[0, :N, :] + p_ref[1, :N, :] + rp_ref[...], 0.0)
        g_ref[...] = _dot_t(h, wr_ref[...])
        r_ref[...] = _dot_t(h, wo_ref[...]) + br_ref[...]

    return pl.pallas_call(
        body,
        out_shape=(jax.ShapeDtypeStruct((N, H), jnp.float32),
                   jax.ShapeDtypeStruct((N, H), jnp.float32)),
    )(p, r_prev, wr, wo, br.reshape(1, H))


def _tc_fin(p, r_prev, lin_w, lin_b):
    def body(p_ref, rp_ref, lw_ref, lb_ref, o_ref):
        h = jnp.maximum(p_ref[0, :N, :] + p_ref[1, :N, :] + rp_ref[...], 0.0)
        emb = jnp.sum(h, axis=0, keepdims=True) * (1.0 / N)
        o_ref[...] = _dot_t(emb, lw_ref[...]) + lb_ref[...]

    c = lin_w.shape[0]
    return pl.pallas_call(
        body,
        out_shape=jax.ShapeDtypeStruct((1, c), jnp.float32),
    )(p, r_prev, lin_w, lin_b.reshape(1, c))


def kernel(x, edge_index, W_rel1, b_rel1, W_root1, W_rel2, b_rel2, W_root2,
           W_rel3, b_rel3, W_root3, lin_W, lin_b):
    src = edge_index[0]
    dst = edge_index[1]

    g1, r1 = _tc_pre(x, W_rel1, W_root1, b_rel1)
    p1 = _sc_segsum(g1, src, dst)
    g2, r2 = _tc_mid(p1, r1, W_rel2, W_root2, b_rel2)
    p2 = _sc_segsum(g2, src, dst)
    g3, r3 = _tc_mid(p2, r2, W_rel3, W_root3, b_rel3)
    p3 = _sc_segsum(g3, src, dst)
    return _tc_fin(p3, r3, lin_W, lin_b)


# speedup vs baseline: 12.7922x; 2.5133x over previous
import functools

import jax
import jax.numpy as jnp
from jax import lax
from jax.experimental import pallas as pl
from jax.experimental.pallas import tpu as pltpu
from jax.experimental.pallas import tpu_sc as plsc

N = 10000
E = 320000
H = 128
NC = 2
NS = 16
NW = NC * NS
EPW = E // NW
K = 125
WINS = EPW // K
NB = 2
NBI = 4
NP = 10112
RPS = NP // NS


def _sc_segsum(g, src, dst, zeros):
    mesh = plsc.VectorSubcoreMesh(core_axis_name="c", subcore_axis_name="s")

    @functools.partial(
        pl.kernel,
        out_type=jax.ShapeDtypeStruct((NC, NP, H), jnp.float32),
        mesh=mesh,
        scratch_types=[
            pltpu.VMEM((NBI, 1, K), jnp.int32),
            pltpu.VMEM((WINS, K), jnp.int32),
            pltpu.VMEM((NB, K, H), jnp.float32),
            pltpu.VMEM_SHARED((NP, H), jnp.float32),
            pltpu.SemaphoreType.DMA((NBI,)),
            pltpu.SemaphoreType.DMA,
            pltpu.SemaphoreType.DMA,
            pltpu.SemaphoreType.DMA((NB,)),
        ],
    )
    def k(g_hbm, src_hbm, dst_hbm, z_hbm, out_hbm, isring, dsts, bufs, acc,
          isems, dsem, zsem, gsems):
        cid = lax.axis_index("c")
        sid = lax.axis_index("s")
        wid = sid * NC + cid
        row0 = sid * RPS
        wbase = wid * WINS

        for i in range(NBI):
            pltpu.async_copy(src_hbm.at[wbase + i], isring.at[i],
                             isems.at[i])
        dc = pltpu.async_copy(dst_hbm.at[wid], dsts, dsem)
        zc = pltpu.async_copy(z_hbm, acc.at[pl.ds(row0, RPS)], zsem)
        dc.wait()
        zc.wait()
        plsc.subcore_barrier()

        for b in range(NB):
            pltpu.make_async_copy(src_hbm.at[wbase + b], isring.at[b],
                                  isems.at[b]).wait()
            pltpu.async_copy(g_hbm.at[isring.at[b].at[0]], bufs.at[b],
                             gsems.at[b])

        @pl.loop(0, WINS, step=NBI)
        def _(w0):
            for u in range(NBI):
                w = w0 + u
                b = u % NB
                pltpu.make_async_copy(g_hbm.at[isring.at[u].at[0]],
                                      bufs.at[b], gsems.at[b]).wait()
                pltpu.sync_copy(bufs.at[b], acc.at[dsts.at[w]], add=True)

                @pl.when(w + NBI < WINS)
                def _():
                    pltpu.async_copy(src_hbm.at[wbase + w + NBI],
                                     isring.at[u], isems.at[u])

                @pl.when(w + NB < WINS)
                def _():
                    u2 = (u + NB) % NBI
                    pltpu.make_async_copy(src_hbm.at[wbase + w + NB],
                                          isring.at[u2], isems.at[u2]).wait()
                    pltpu.async_copy(g_hbm.at[isring.at[u2].at[0]],
                                     bufs.at[b], gsems.at[b])

        plsc.subcore_barrier()
        pltpu.sync_copy(acc.at[pl.ds(row0, RPS)],
                        out_hbm.at[cid].at[pl.ds(row0, RPS)])

    return k(g, src, dst, zeros)


def _dot_t(a, b):
    return lax.dot_general(a, b, (((1,), (1,)), ((), ())),
                           preferred_element_type=jnp.float32)


def _tc_pre(x, wr, wo, br):
    def body(x_ref, wr_ref, wo_ref, br_ref, g_ref, r_ref):
        xv = x_ref[...]
        g_ref[...] = _dot_t(xv, wr_ref[...])
        r_ref[...] = _dot_t(xv, wo_ref[...]) + br_ref[...]

    return pl.pallas_call(
        body,
        out_shape=(jax.ShapeDtypeStruct((N, H), jnp.float32),
                   jax.ShapeDtypeStruct((N, H), jnp.float32)),
    )(x, wr, wo, br.reshape(1, H))


def _tc_mid(p, r_prev, wr, wo, br):
    def body(p_ref, rp_ref, wr_ref, wo_ref, br_ref, g_ref, r_ref):
        h = jnp.maximum(p_ref[0, :N, :] + p_ref[1, :N, :] + rp_ref[...], 0.0)
        g_ref[...] = _dot_t(h, wr_ref[...])
        r_ref[...] = _dot_t(h, wo_ref[...]) + br_ref[...]

    return pl.pallas_call(
        body,
        out_shape=(jax.ShapeDtypeStruct((N, H), jnp.float32),
                   jax.ShapeDtypeStruct((N, H), jnp.float32)),
    )(p, r_prev, wr, wo, br.reshape(1, H))


def _tc_fin(p, r_prev, lin_w, lin_b):
    def body(p_ref, rp_ref, lw_ref, lb_ref, o_ref):
        h = jnp.maximum(p_ref[0, :N, :] + p_ref[1, :N, :] + rp_ref[...], 0.0)
        emb = jnp.sum(h, axis=0, keepdims=True) * (1.0 / N)
        o_ref[...] = _dot_t(emb, lw_ref[...]) + lb_ref[...]

    c = lin_w.shape[0]
    return pl.pallas_call(
        body,
        out_shape=jax.ShapeDtypeStruct((1, c), jnp.float32),
    )(p, r_prev, lin_w, lin_b.reshape(1, c))


def kernel(x, edge_index, W_rel1, b_rel1, W_root1, W_rel2, b_rel2, W_root2,
           W_rel3, b_rel3, W_root3, lin_W, lin_b):
    src = edge_index[0].reshape(NW * WINS, 1, K)
    dst = edge_index[1].reshape(NW, WINS, K)
    zeros = jnp.zeros((RPS, H), jnp.float32)

    g1, r1 = _tc_pre(x, W_rel1, W_root1, b_rel1)
    p1 = _sc_segsum(g1, src, dst, zeros)
    g2, r2 = _tc_mid(p1, r1, W_rel2, W_root2, b_rel2)
    p2 = _sc_segsum(g2, src, dst, zeros)
    g3, r3 = _tc_mid(p2, r2, W_rel3, W_root3, b_rel3)
    p3 = _sc_segsum(g3, src, dst, zeros)
    return _tc_fin(p3, r3, lin_W, lin_b)


# speedup vs baseline: 12.8370x; 1.0035x over previous
import functools

import jax
import jax.numpy as jnp
from jax import lax
from jax.experimental import pallas as pl
from jax.experimental.pallas import tpu as pltpu
from jax.experimental.pallas import tpu_sc as plsc

N = 10000
E = 320000
H = 128
NC = 2
NS = 16
NW = NC * NS
EPW = E // NW
K = 125
WINS = EPW // K
NB = 2
NBI = 4
NP = 10112
RPS = NP // NS


def _sc_segsum(g, src, dst, zeros):
    mesh = plsc.VectorSubcoreMesh(core_axis_name="c", subcore_axis_name="s")

    @functools.partial(
        pl.kernel,
        out_type=jax.ShapeDtypeStruct((NC, NP, H), jnp.float32),
        mesh=mesh,
        scratch_types=[
            pltpu.VMEM((NBI, 1, K), jnp.int32),
            pltpu.VMEM((WINS, K), jnp.int32),
            pltpu.VMEM((NB, K, H), jnp.float32),
            pltpu.VMEM_SHARED((NP, H), jnp.float32),
            pltpu.SemaphoreType.DMA((NBI,)),
            pltpu.SemaphoreType.DMA,
            pltpu.SemaphoreType.DMA,
            pltpu.SemaphoreType.DMA((NB,)),
            pltpu.SemaphoreType.DMA((NB,)),
        ],
    )
    def k(g_hbm, src_hbm, dst_hbm, z_hbm, out_hbm, isring, dsts, bufs, acc,
          isems, dsem, zsem, gsems, ssems):
        cid = lax.axis_index("c")
        sid = lax.axis_index("s")
        wid = sid * NC + cid
        row0 = sid * RPS
        wbase = wid * WINS

        for i in range(NBI):
            pltpu.async_copy(src_hbm.at[wbase + i], isring.at[i],
                             isems.at[i])
        dc = pltpu.async_copy(dst_hbm.at[wid], dsts, dsem)
        zc = pltpu.async_copy(z_hbm, acc.at[pl.ds(row0, RPS)], zsem)
        dc.wait()
        zc.wait()
        plsc.subcore_barrier()

        pltpu.make_async_copy(src_hbm.at[wbase], isring.at[0],
                              isems.at[0]).wait()
        pltpu.async_copy(g_hbm.at[isring.at[0].at[0]], bufs.at[0],
                         gsems.at[0])

        @pl.loop(0, WINS, step=NBI)
        def _(w0):
            for u in range(NBI):
                w = w0 + u
                b = u % NB
                bp = 1 - b
                up = (u + 1) % NBI

                @pl.when(w + 1 < WINS)
                def _():
                    @pl.when(w >= 1)
                    def _():
                        pltpu.make_async_copy(bufs.at[bp],
                                              acc.at[dsts.at[w]],
                                              ssems.at[bp]).wait()
                    pltpu.make_async_copy(src_hbm.at[wbase + w + 1],
                                          isring.at[up], isems.at[up]).wait()
                    pltpu.async_copy(g_hbm.at[isring.at[up].at[0]],
                                     bufs.at[bp], gsems.at[bp])

                pltpu.make_async_copy(g_hbm.at[isring.at[u].at[0]],
                                      bufs.at[b], gsems.at[b]).wait()
                pltpu.async_copy(bufs.at[b], acc.at[dsts.at[w]],
                                 ssems.at[b], add=True)

                @pl.when(w + NBI < WINS)
                def _():
                    pltpu.async_copy(src_hbm.at[wbase + w + NBI],
                                     isring.at[u], isems.at[u])

        for b in range(NB):
            pltpu.make_async_copy(bufs.at[b], acc.at[dsts.at[0]],
                                  ssems.at[b]).wait()
        plsc.subcore_barrier()
        pltpu.sync_copy(acc.at[pl.ds(row0, RPS)],
                        out_hbm.at[cid].at[pl.ds(row0, RPS)])

    return k(g, src, dst, zeros)


def _dot_t(a, b):
    return lax.dot_general(a, b, (((1,), (1,)), ((), ())),
                           preferred_element_type=jnp.float32)


def _tc_pre(x, wr, wo, br):
    def body(x_ref, wr_ref, wo_ref, br_ref, g_ref, r_ref):
        xv = x_ref[...]
        g_ref[...] = _dot_t(xv, wr_ref[...])
        r_ref[...] = _dot_t(xv, wo_ref[...]) + br_ref[...]

    return pl.pallas_call(
        body,
        out_shape=(jax.ShapeDtypeStruct((N, H), jnp.float32),
                   jax.ShapeDtypeStruct((N, H), jnp.float32)),
    )(x, wr, wo, br.reshape(1, H))


def _tc_mid(p, r_prev, wr, wo, br):
    def body(p_ref, rp_ref, wr_ref, wo_ref, br_ref, g_ref, r_ref):
        h = jnp.maximum(p_ref[0, :N, :] + p_ref[1, :N, :] + rp_ref[...], 0.0)
        g_ref[...] = _dot_t(h, wr_ref[...])
        r_ref[...] = _dot_t(h, wo_ref[...]) + br_ref[...]

    return pl.pallas_call(
        body,
        out_shape=(jax.ShapeDtypeStruct((N, H), jnp.float32),
                   jax.ShapeDtypeStruct((N, H), jnp.float32)),
    )(p, r_prev, wr, wo, br.reshape(1, H))


def _tc_fin(p, r_prev, lin_w, lin_b):
    def body(p_ref, rp_ref, lw_ref, lb_ref, o_ref):
        h = jnp.maximum(p_ref[0, :N, :] + p_ref[1, :N, :] + rp_ref[...], 0.0)
        emb = jnp.sum(h, axis=0, keepdims=True) * (1.0 / N)
        o_ref[...] = _dot_t(emb, lw_ref[...]) + lb_ref[...]

    c = lin_w.shape[0]
    return pl.pallas_call(
        body,
        out_shape=jax.ShapeDtypeStruct((1, c), jnp.float32),
    )(p, r_prev, lin_w, lin_b.reshape(1, c))


def kernel(x, edge_index, W_rel1, b_rel1, W_root1, W_rel2, b_rel2, W_root2,
           W_rel3, b_rel3, W_root3, lin_W, lin_b):
    src = edge_index[0].reshape(NW * WINS, 1, K)
    dst = edge_index[1].reshape(NW, WINS, K)
    zeros = jnp.zeros((RPS, H), jnp.float32)

    g1, r1 = _tc_pre(x, W_rel1, W_root1, b_rel1)
    p1 = _sc_segsum(g1, src, dst, zeros)
    g2, r2 = _tc_mid(p1, r1, W_rel2, W_root2, b_rel2)
    p2 = _sc_segsum(g2, src, dst, zeros)
    g3, r3 = _tc_mid(p2, r2, W_rel3, W_root3, b_rel3)
    p3 = _sc_segsum(g3, src, dst, zeros)
    return _tc_fin(p3, r3, lin_W, lin_b)


# speedup vs baseline: 12.9269x; 1.0070x over previous
import functools

import jax
import jax.numpy as jnp
from jax import lax
from jax.experimental import pallas as pl
from jax.experimental.pallas import tpu as pltpu
from jax.experimental.pallas import tpu_sc as plsc

N = 10000
E = 320000
H = 128
NC = 2
NS = 16
NW = NC * NS
EPW = E // NW
K = 125
WINS = EPW // K
NB = 2
NBI = 4
NP = 10112
RPS = NP // NS


def _sc_segsum(g, src, dst, zeros):
    mesh = plsc.VectorSubcoreMesh(core_axis_name="c", subcore_axis_name="s")

    @functools.partial(
        pl.kernel,
        out_type=jax.ShapeDtypeStruct((NC, NP, H), jnp.float32),
        mesh=mesh,
        scratch_types=[
            pltpu.VMEM((NBI, 1, K), jnp.int32),
            pltpu.VMEM((WINS, K), jnp.int32),
            pltpu.VMEM((NB, K, H), jnp.float32),
            pltpu.VMEM_SHARED((NP, H), jnp.float32),
            pltpu.SemaphoreType.DMA((NBI,)),
            pltpu.SemaphoreType.DMA,
            pltpu.SemaphoreType.DMA,
            pltpu.SemaphoreType.DMA((NB,)),
            pltpu.SemaphoreType.DMA((NB,)),
        ],
    )
    def k(g_hbm, src_hbm, dst_hbm, z_hbm, out_hbm, isring, dsts, bufs, acc,
          isems, dsem, zsem, gsems, ssems):
        cid = lax.axis_index("c")
        sid = lax.axis_index("s")
        wid = sid * NC + cid
        row0 = sid * RPS
        wbase = wid * WINS

        for i in range(NBI):
            pltpu.async_copy(src_hbm.at[wbase + i], isring.at[i],
                             isems.at[i])
        dc = pltpu.async_copy(dst_hbm.at[wid], dsts, dsem)
        zc = pltpu.async_copy(z_hbm, acc.at[pl.ds(row0, RPS)], zsem)

        pltpu.make_async_copy(src_hbm.at[wbase], isring.at[0],
                              isems.at[0]).wait()
        pltpu.async_copy(g_hbm.at[isring.at[0].at[0]], bufs.at[0],
                         gsems.at[0])
        dc.wait()
        zc.wait()
        plsc.subcore_barrier()

        @pl.loop(0, WINS, step=NBI)
        def _(w0):
            for u in range(NBI):
                w = w0 + u
                b = u % NB
                bp = 1 - b
                up = (u + 1) % NBI

                @pl.when(w + 1 < WINS)
                def _():
                    @pl.when(w >= 1)
                    def _():
                        pltpu.make_async_copy(bufs.at[bp],
                                              acc.at[dsts.at[w]],
                                              ssems.at[bp]).wait()
                    pltpu.make_async_copy(src_hbm.at[wbase + w + 1],
                                          isring.at[up], isems.at[up]).wait()
                    pltpu.async_copy(g_hbm.at[isring.at[up].at[0]],
                                     bufs.at[bp], gsems.at[bp])

                pltpu.make_async_copy(g_hbm.at[isring.at[u].at[0]],
                                      bufs.at[b], gsems.at[b]).wait()
                pltpu.async_copy(bufs.at[b], acc.at[dsts.at[w]],
                                 ssems.at[b], add=True)

                @pl.when(w + NBI < WINS)
                def _():
                    pltpu.async_copy(src_hbm.at[wbase + w + NBI],
                                     isring.at[u], isems.at[u])

        for b in range(NB):
            pltpu.make_async_copy(bufs.at[b], acc.at[dsts.at[0]],
                                  ssems.at[b]).wait()
        plsc.subcore_barrier()
        pltpu.sync_copy(acc.at[pl.ds(row0, RPS)],
                        out_hbm.at[cid].at[pl.ds(row0, RPS)])

    return k(g, src, dst, zeros)


def _dot_t(a, b):
    return lax.dot_general(a, b, (((1,), (1,)), ((), ())),
                           preferred_element_type=jnp.float32)


def _tc_pre(x, wr, wo, br):
    def body(x_ref, wr_ref, wo_ref, br_ref, g_ref, r_ref):
        xv = x_ref[...]
        g_ref[...] = _dot_t(xv, wr_ref[...])
        r_ref[...] = _dot_t(xv, wo_ref[...]) + br_ref[...]

    return pl.pallas_call(
        body,
        out_shape=(jax.ShapeDtypeStruct((N, H), jnp.float32),
                   jax.ShapeDtypeStruct((N, H), jnp.float32)),
    )(x, wr, wo, br.reshape(1, H))


def _tc_mid(p, r_prev, wr, wo, br):
    def body(p_ref, rp_ref, wr_ref, wo_ref, br_ref, g_ref, r_ref):
        h = jnp.maximum(p_ref[0, :N, :] + p_ref[1, :N, :] + rp_ref[...], 0.0)
        g_ref[...] = _dot_t(h, wr_ref[...])
        r_ref[...] = _dot_t(h, wo_ref[...]) + br_ref[...]

    return pl.pallas_call(
        body,
        out_shape=(jax.ShapeDtypeStruct((N, H), jnp.float32),
                   jax.ShapeDtypeStruct((N, H), jnp.float32)),
    )(p, r_prev, wr, wo, br.reshape(1, H))


def _tc_fin(p, r_prev, lin_w, lin_b):
    def body(p_ref, rp_ref, lw_ref, lb_ref, o_ref):
        h = jnp.maximum(p_ref[0, :N, :] + p_ref[1, :N, :] + rp_ref[...], 0.0)
        emb = jnp.sum(h, axis=0, keepdims=True) * (1.0 / N)
        o_ref[...] = _dot_t(emb, lw_ref[...]) + lb_ref[...]

    c = lin_w.shape[0]
    return pl.pallas_call(
        body,
        out_shape=jax.ShapeDtypeStruct((1, c), jnp.float32),
    )(p, r_prev, lin_w, lin_b.reshape(1, c))


def kernel(x, edge_index, W_rel1, b_rel1, W_root1, W_rel2, b_rel2, W_root2,
           W_rel3, b_rel3, W_root3, lin_W, lin_b):
    src = edge_index[0].reshape(NW * WINS, 1, K)
    dst = edge_index[1].reshape(NW, WINS, K)
    zeros = jnp.zeros((RPS, H), jnp.float32)

    g1, r1 = _tc_pre(x, W_rel1, W_root1, b_rel1)
    p1 = _sc_segsum(g1, src, dst, zeros)
    g2, r2 = _tc_mid(p1, r1, W_rel2, W_root2, b_rel2)
    p2 = _sc_segsum(g2, src, dst, zeros)
    g3, r3 = _tc_mid(p2, r2, W_rel3, W_root3, b_rel3)
    p3 = _sc_segsum(g3, src, dst, zeros)
    return _tc_fin(p3, r3, lin_W, lin_b)
